# Initial kernel scaffold; baseline (speedup 1.0000x reference)
import jax
import jax.numpy as jnp
from jax.experimental import pallas as pl


def kernel(query, Wg, bg, Wn, bn, ws, bs, wc, bc, W1, b1, gamma, beta, W2, b2):
    raise NotImplementedError("write your pallas kernel here")


# speedup vs baseline: 2.7507x; 2.7507x over previous
import functools

import jax
import jax.numpy as jnp
from jax.experimental import pallas as pl
from jax.experimental.pallas import tpu as pltpu

N_TOKENS = 32768
D = 768
H = 256
NE = 3
BLK = 512


def _fused_body(nblk, q_ref, wg_ref, bg_ref, bn_ref, ws_ref, bs_ref, wc_ref,
                bc_ref, w1_ref, b1_ref, gamma_ref, beta_ref, w2_ref, b2_ref,
                out_ref, sw_ref, loss_ref, acc_ref):
    i = pl.program_id(0)

    q = q_ref[...]
    wg = wg_ref[...]
    logits = jnp.dot(q, wg, preferred_element_type=jnp.float32)
    logits = logits + bg_ref[...][None, :]

    l0 = logits[:, 0:1]
    l1 = logits[:, 1:2]
    l2 = logits[:, 2:3]

    j2 = (l2 <= l0) & (l2 <= l1)
    j1 = (~j2) & (l1 <= l0)
    j0 = ~(j2 | j1)

    m = jnp.maximum(l0, jnp.maximum(l1, l2))
    e0 = jnp.exp(l0 - m)
    e1 = jnp.exp(l1 - m)
    e2 = jnp.exp(l2 - m)

    t = e0 + e1 + e2
    p0 = e0 / t
    p1 = e1 / t
    p2 = e2 / t

    z0 = jnp.where(j0, 0.0, e0)
    z1 = jnp.where(j1, 0.0, e1)
    z2 = jnp.where(j2, 0.0, e2)
    s = z0 + z1 + z2
    w0 = z0 / s
    w1 = z1 / s
    w2 = z2 / s

    sw = jnp.concatenate([w0, w1, w2], axis=1)
    sw_ref[...] = sw

    @pl.when(i == 0)
    def _init():
        for k in range(8):
            acc_ref[k] = 0.0

    acc_ref[0] += jnp.sum(p0)
    acc_ref[1] += jnp.sum(p1)
    acc_ref[2] += jnp.sum(p2)
    acc_ref[3] += jnp.sum(jnp.where(j0, 0.0, 1.0))
    acc_ref[4] += jnp.sum(jnp.where(j1, 0.0, 1.0))
    acc_ref[5] += jnp.sum(jnp.where(j2, 0.0, 1.0))

    neural = bn_ref[...][None, :]
    symbolic = (0.5 * ws_ref[...] + bs_ref[...])[None, :]
    categorical = (0.5 * wc_ref[...] + bc_ref[...])[None, :]
    experts = jnp.concatenate([neural, symbolic, categorical], axis=0)
    e1t = jnp.dot(experts, w1_ref[...], preferred_element_type=jnp.float32)

    hpre = jnp.dot(sw, e1t, preferred_element_type=jnp.float32)
    hpre = hpre + b1_ref[...][None, :]

    g = 0.5 * hpre * (1.0 + jax.lax.erf(hpre * 0.7071067811865476))

    mu = jnp.mean(g, axis=1, keepdims=True)
    var = jnp.mean((g - mu) * (g - mu), axis=1, keepdims=True)
    hn = (g - mu) * jax.lax.rsqrt(var + 1e-5)
    hn = hn * gamma_ref[...][None, :] + beta_ref[...][None, :]

    out = jnp.dot(hn, w2_ref[...], preferred_element_type=jnp.float32)
    out_ref[...] = out + b2_ref[...][None, :]

    @pl.when(i == nblk - 1)
    def _finalize():
        inv_b = 1.0 / N_TOKENS
        loss = 0.0
        for k in range(NE):
            loss += (acc_ref[k] * inv_b) * (acc_ref[k + 3] * inv_b)
        loss_ref[0] = NE * loss


def kernel(query, Wg, bg, Wn, bn, ws, bs, wc, bc, W1, b1, gamma, beta, W2, b2):
    nblk = N_TOKENS // BLK
    full = lambda shape: pl.BlockSpec(shape, lambda i: tuple(0 for _ in shape))
    out, sw, loss = pl.pallas_call(
        functools.partial(_fused_body, nblk),
        grid=(nblk,),
        in_specs=[
            pl.BlockSpec((BLK, D), lambda i: (i, 0)),
            full((D, NE)), full((NE,)), full((D,)), full((D,)), full((D,)),
            full((D,)), full((D,)), full((D, H)), full((H,)), full((H,)),
            full((H,)), full((H, D)), full((D,)),
        ],
        out_specs=[
            pl.BlockSpec((BLK, D), lambda i: (i, 0)),
            pl.BlockSpec((BLK, NE), lambda i: (i, 0)),
            pl.BlockSpec(memory_space=pltpu.SMEM),
        ],
        out_shape=[
            jax.ShapeDtypeStruct((N_TOKENS, D), jnp.float32),
            jax.ShapeDtypeStruct((N_TOKENS, NE), jnp.float32),
            jax.ShapeDtypeStruct((1,), jnp.float32),
        ],
        scratch_shapes=[pltpu.SMEM((8,), jnp.float32)],
    )(query, Wg, bg, bn, ws, bs, wc, bc, W1, b1, gamma, beta, W2, b2)
    return out, sw, loss[0]


# speedup vs baseline: 2.7874x; 1.0133x over previous
import functools

import jax
import jax.numpy as jnp
from jax import lax
from jax.experimental import pallas as pl
from jax.experimental.pallas import tpu as pltpu
from jax.experimental.pallas import tpu_sc as plsc

N_TOKENS = 32768
D = 768
H = 256
NE = 3

NC = 2
NS = 16
NW = NC * NS
LANES = 16
CHUNK = N_TOKENS // NW
PART_W = 6 * LANES

BLK_A = 2048
BLK_C = 512


def _logits_body(q_ref, wg_ref, bg_ref, lg_ref):
    lg = jnp.dot(q_ref[...], wg_ref[...], preferred_element_type=jnp.float32)
    lg_ref[...] = lg + bg_ref[...][None, :]


def _logits_call(query, Wg, bg):
    nblk = N_TOKENS // BLK_A
    return pl.pallas_call(
        _logits_body,
        grid=(nblk,),
        in_specs=[
            pl.BlockSpec((BLK_A, D), lambda i: (i, 0)),
            pl.BlockSpec((D, NE), lambda i: (0, 0)),
            pl.BlockSpec((NE,), lambda i: (0,)),
        ],
        out_specs=pl.BlockSpec((BLK_A, NE), lambda i: (i, 0)),
        out_shape=jax.ShapeDtypeStruct((N_TOKENS, NE), jnp.float32),
    )(query, Wg, bg)


def _gate_body(lg_hbm, sw_hbm, part_hbm, lv, wv, pv):
    wid = lax.axis_index("s") * NC + lax.axis_index("c")
    base = wid * CHUNK * NE
    pltpu.sync_copy(lg_hbm.at[pl.ds(base, CHUNK * NE)], lv)

    lane = lax.broadcasted_iota(jnp.int32, (LANES,), 0)
    fzero = jnp.zeros((LANES,), jnp.float32)
    fone = jnp.full((LANES,), 1.0, jnp.float32)

    def step(j, acc):
        i0, i1, i2, c0, c1, c2 = acc
        pos = (j * LANES + lane) * NE
        l0 = plsc.load_gather(lv, [pos])
        l1 = plsc.load_gather(lv, [pos + 1])
        l2 = plsc.load_gather(lv, [pos + 2])

        j2 = (l2 <= l0) & (l2 <= l1)
        j1 = (~j2) & (l1 <= l0)
        j0 = ~(j2 | j1)

        m = jnp.maximum(l0, jnp.maximum(l1, l2))
        e0 = jnp.exp(l0 - m)
        e1 = jnp.exp(l1 - m)
        e2 = jnp.exp(l2 - m)
        rt = fone / (e0 + e1 + e2)

        z0 = jnp.where(j0, fzero, e0)
        z1 = jnp.where(j1, fzero, e1)
        z2 = jnp.where(j2, fzero, e2)
        rs = fone / (z0 + z1 + z2)
        plsc.store_scatter(wv, [pos], z0 * rs)
        plsc.store_scatter(wv, [pos + 1], z1 * rs)
        plsc.store_scatter(wv, [pos + 2], z2 * rs)

        return (i0 + e0 * rt, i1 + e1 * rt, i2 + e2 * rt,
                c0 + jnp.where(j0, fzero, fone),
                c1 + jnp.where(j1, fzero, fone),
                c2 + jnp.where(j2, fzero, fone))

    init = (fzero, fzero, fzero, fzero, fzero, fzero)
    acc = lax.fori_loop(0, CHUNK // LANES, step, init)
    for k in range(6):
        pv[pl.ds(k * LANES, LANES)] = acc[k]

    pltpu.sync_copy(wv, sw_hbm.at[pl.ds(base, CHUNK * NE)])
    pltpu.sync_copy(pv, part_hbm.at[wid])


def _gate_call(lg_flat):
    mesh = plsc.VectorSubcoreMesh(
        core_axis_name="c", subcore_axis_name="s",
        num_cores=NC, num_subcores=NS)
    f = pl.kernel(
        _gate_body,
        out_type=[
            jax.ShapeDtypeStruct((N_TOKENS * NE,), jnp.float32),
            jax.ShapeDtypeStruct((NW, PART_W), jnp.float32),
        ],
        mesh=mesh,
        scratch_types=[
            pltpu.VMEM((CHUNK * NE,), jnp.float32),
            pltpu.VMEM((CHUNK * NE,), jnp.float32),
            pltpu.VMEM((PART_W,), jnp.float32),
        ],
        compiler_params=pltpu.CompilerParams(needs_layout_passes=False),
    )
    return f(lg_flat)


def _head_body(nblk, sw_ref, part_ref, bn_ref, ws_ref, bs_ref, wc_ref, bc_ref,
               w1_ref, b1_ref, gamma_ref, beta_ref, w2_ref, b2_ref,
               out_ref, loss_ref, e1_ref):
    i = pl.program_id(0)

    @pl.when(i == 0)
    def _prep():
        neural = bn_ref[...][None, :]
        symbolic = (0.5 * ws_ref[...] + bs_ref[...])[None, :]
        categorical = (0.5 * wc_ref[...] + bc_ref[...])[None, :]
        experts = jnp.concatenate([neural, symbolic, categorical], axis=0)
        e1_ref[...] = jnp.dot(experts, w1_ref[...],
                              preferred_element_type=jnp.float32)

        p = part_ref[...]
        inv_b = 1.0 / N_TOKENS
        loss = 0.0
        for e in range(NE):
            imp = jnp.sum(p[:, e * LANES:(e + 1) * LANES])
            cnt = jnp.sum(p[:, (NE + e) * LANES:(NE + e + 1) * LANES])
            loss += (imp * inv_b) * (cnt * inv_b)
        loss_ref[0] = NE * loss

    hpre = jnp.dot(sw_ref[...], e1_ref[...],
                   preferred_element_type=jnp.float32)
    hpre = hpre + b1_ref[...][None, :]

    g = 0.5 * hpre * (1.0 + lax.erf(hpre * 0.7071067811865476))

    mu = jnp.mean(g, axis=1, keepdims=True)
    var = jnp.mean((g - mu) * (g - mu), axis=1, keepdims=True)
    hn = (g - mu) * lax.rsqrt(var + 1e-5)
    hn = hn * gamma_ref[...][None, :] + beta_ref[...][None, :]

    out = jnp.dot(hn, w2_ref[...], preferred_element_type=jnp.float32)
    out_ref[...] = out + b2_ref[...][None, :]


def _head_call(sw, part, bn, ws, bs, wc, bc, W1, b1, gamma, beta, W2, b2):
    nblk = N_TOKENS // BLK_C
    full = lambda shape: pl.BlockSpec(shape, lambda i: tuple(0 for _ in shape))
    return pl.pallas_call(
        functools.partial(_head_body, nblk),
        grid=(nblk,),
        in_specs=[
            pl.BlockSpec((BLK_C, NE), lambda i: (i, 0)),
            full((NW, PART_W)), full((D,)), full((D,)), full((D,)),
            full((D,)), full((D,)), full((D, H)), full((H,)), full((H,)),
            full((H,)), full((H, D)), full((D,)),
        ],
        out_specs=[
            pl.BlockSpec((BLK_C, D), lambda i: (i, 0)),
            pl.BlockSpec(memory_space=pltpu.SMEM),
        ],
        out_shape=[
            jax.ShapeDtypeStruct((N_TOKENS, D), jnp.float32),
            jax.ShapeDtypeStruct((1,), jnp.float32),
        ],
        scratch_shapes=[pltpu.VMEM((NE, H), jnp.float32)],
    )(sw, part, bn, ws, bs, wc, bc, W1, b1, gamma, beta, W2, b2)


def kernel(query, Wg, bg, Wn, bn, ws, bs, wc, bc, W1, b1, gamma, beta, W2, b2):
    lg = _logits_call(query, Wg, bg)
    sw_flat, part = _gate_call(lg.reshape(-1))
    sw = sw_flat.reshape(N_TOKENS, NE)
    out, loss = _head_call(sw, part, bn, ws, bs, wc, bc,
                           W1, b1, gamma, beta, W2, b2)
    return out, sw, loss[0]


# speedup vs baseline: 3.0824x; 1.1058x over previous
import functools

import jax
import jax.numpy as jnp
from jax import lax
from jax.experimental import pallas as pl
from jax.experimental.pallas import tpu as pltpu
from jax.experimental.pallas import tpu_sc as plsc

N_TOKENS = 32768
D = 768
H = 256
NE = 3

NC = 2
NS = 16
NW = NC * NS
LANES = 16
CHUNK = N_TOKENS // NW
PART_W = 6 * LANES

BLK_A = 2048
BLK_C = 1024


def _logits_body(q_ref, wg_ref, bg_ref, lg_ref):
    lg = jnp.dot(q_ref[...], wg_ref[...], preferred_element_type=jnp.float32)
    lg_ref[...] = lg + bg_ref[...][None, :]


def _logits_call(query, Wg, bg):
    nblk = N_TOKENS // BLK_A
    return pl.pallas_call(
        _logits_body,
        grid=(nblk,),
        in_specs=[
            pl.BlockSpec((BLK_A, D), lambda i: (i, 0)),
            pl.BlockSpec((D, NE), lambda i: (0, 0)),
            pl.BlockSpec((NE,), lambda i: (0,)),
        ],
        out_specs=pl.BlockSpec((BLK_A, NE), lambda i: (i, 0)),
        out_shape=jax.ShapeDtypeStruct((N_TOKENS, NE), jnp.float32),
    )(query, Wg, bg)


def _gate_body(lg_hbm, sw_hbm, part_hbm, lv, wv, pv):
    wid = lax.axis_index("s") * NC + lax.axis_index("c")
    base = wid * CHUNK * NE
    pltpu.sync_copy(lg_hbm.at[pl.ds(base, CHUNK * NE)], lv)

    lane = lax.broadcasted_iota(jnp.int32, (LANES,), 0)
    fzero = jnp.zeros((LANES,), jnp.float32)
    fone = jnp.full((LANES,), 1.0, jnp.float32)

    def step(j, acc):
        i0, i1, i2, c0, c1, c2 = acc
        pos = (j * LANES + lane) * NE
        l0 = plsc.load_gather(lv, [pos])
        l1 = plsc.load_gather(lv, [pos + 1])
        l2 = plsc.load_gather(lv, [pos + 2])

        j2 = (l2 <= l0) & (l2 <= l1)
        j1 = (~j2) & (l1 <= l0)
        j0 = ~(j2 | j1)

        m = jnp.maximum(l0, jnp.maximum(l1, l2))
        e0 = jnp.exp(l0 - m)
        e1 = jnp.exp(l1 - m)
        e2 = jnp.exp(l2 - m)
        rt = fone / (e0 + e1 + e2)

        z0 = jnp.where(j0, fzero, e0)
        z1 = jnp.where(j1, fzero, e1)
        z2 = jnp.where(j2, fzero, e2)
        rs = fone / (z0 + z1 + z2)
        plsc.store_scatter(wv, [pos], z0 * rs)
        plsc.store_scatter(wv, [pos + 1], z1 * rs)
        plsc.store_scatter(wv, [pos + 2], z2 * rs)

        return (i0 + e0 * rt, i1 + e1 * rt, i2 + e2 * rt,
                c0 + jnp.where(j0, fzero, fone),
                c1 + jnp.where(j1, fzero, fone),
                c2 + jnp.where(j2, fzero, fone))

    init = (fzero, fzero, fzero, fzero, fzero, fzero)
    acc = lax.fori_loop(0, CHUNK // LANES, step, init)
    for k in range(6):
        pv[pl.ds(k * LANES, LANES)] = acc[k]

    pltpu.sync_copy(wv, sw_hbm.at[pl.ds(base, CHUNK * NE)])
    pltpu.sync_copy(pv, part_hbm.at[wid])


def _gate_call(lg_flat):
    mesh = plsc.VectorSubcoreMesh(
        core_axis_name="c", subcore_axis_name="s",
        num_cores=NC, num_subcores=NS)
    f = pl.kernel(
        _gate_body,
        out_type=[
            jax.ShapeDtypeStruct((N_TOKENS * NE,), jnp.float32),
            jax.ShapeDtypeStruct((NW, PART_W), jnp.float32),
        ],
        mesh=mesh,
        scratch_types=[
            pltpu.VMEM((CHUNK * NE,), jnp.float32),
            pltpu.VMEM((CHUNK * NE,), jnp.float32),
            pltpu.VMEM((PART_W,), jnp.float32),
        ],
        compiler_params=pltpu.CompilerParams(needs_layout_passes=False),
    )
    return f(lg_flat)


def _head_body(nblk, sw_ref, part_ref, bn_ref, ws_ref, bs_ref, wc_ref, bc_ref,
               w1_ref, b1_ref, gamma_ref, beta_ref, w2_ref, b2_ref,
               out_ref, loss_ref, e1_ref):
    i = pl.program_id(0)

    @pl.when(i == 0)
    def _prep():
        neural = bn_ref[...][None, :]
        symbolic = (0.5 * ws_ref[...] + bs_ref[...])[None, :]
        categorical = (0.5 * wc_ref[...] + bc_ref[...])[None, :]
        experts = jnp.concatenate([neural, symbolic, categorical], axis=0)
        e1_ref[...] = jnp.dot(experts, w1_ref[...],
                              preferred_element_type=jnp.float32)

        p = part_ref[...]
        inv_b = 1.0 / N_TOKENS
        loss = 0.0
        for e in range(NE):
            imp = jnp.sum(p[:, e * LANES:(e + 1) * LANES])
            cnt = jnp.sum(p[:, (NE + e) * LANES:(NE + e + 1) * LANES])
            loss += (imp * inv_b) * (cnt * inv_b)
        loss_ref[0] = NE * loss

    hpre = jnp.dot(sw_ref[...], e1_ref[...],
                   preferred_element_type=jnp.float32)
    hpre = hpre + b1_ref[...][None, :]

    g = 0.5 * hpre * (1.0 + lax.erf(hpre * 0.7071067811865476))

    mu = jnp.mean(g, axis=1, keepdims=True)
    var = jnp.mean((g - mu) * (g - mu), axis=1, keepdims=True)
    hn = (g - mu) * lax.rsqrt(var + 1e-5)
    hn = hn * gamma_ref[...][None, :] + beta_ref[...][None, :]

    out = jnp.dot(hn.astype(jnp.bfloat16), w2_ref[...],
                  preferred_element_type=jnp.float32)
    out_ref[...] = out + b2_ref[...][None, :]


def _head_call(sw, part, bn, ws, bs, wc, bc, W1, b1, gamma, beta, W2, b2):
    nblk = N_TOKENS // BLK_C
    full = lambda shape: pl.BlockSpec(shape, lambda i: tuple(0 for _ in shape))
    return pl.pallas_call(
        functools.partial(_head_body, nblk),
        grid=(nblk,),
        in_specs=[
            pl.BlockSpec((BLK_C, NE), lambda i: (i, 0)),
            full((NW, PART_W)), full((D,)), full((D,)), full((D,)),
            full((D,)), full((D,)), full((D, H)), full((H,)), full((H,)),
            full((H,)), full((H, D)), full((D,)),
        ],
        out_specs=[
            pl.BlockSpec((BLK_C, D), lambda i: (i, 0)),
            pl.BlockSpec(memory_space=pltpu.SMEM),
        ],
        out_shape=[
            jax.ShapeDtypeStruct((N_TOKENS, D), jnp.float32),
            jax.ShapeDtypeStruct((1,), jnp.float32),
        ],
        scratch_shapes=[pltpu.VMEM((NE, H), jnp.float32)],
    )(sw, part, bn, ws, bs, wc, bc, W1, b1, gamma, beta,
      W2.astype(jnp.bfloat16), b2)


def kernel(query, Wg, bg, Wn, bn, ws, bs, wc, bc, W1, b1, gamma, beta, W2, b2):
    lg = _logits_call(query, Wg, bg)
    sw_flat, part = _gate_call(lg.reshape(-1))
    sw = sw_flat.reshape(N_TOKENS, NE)
    out, loss = _head_call(sw, part, bn, ws, bs, wc, bc,
                           W1, b1, gamma, beta, W2, b2)
    return out, sw, loss[0]


# speedup vs baseline: 4.3522x; 1.4120x over previous
import functools

import jax
import jax.numpy as jnp
from jax import lax
from jax.experimental import pallas as pl
from jax.experimental.pallas import tpu as pltpu
from jax.experimental.pallas import tpu_sc as plsc

N_TOKENS = 32768
D = 768
H = 256
NE = 3

NC = 2
NS = 16
NW = NC * NS
LANES = 16
CHUNK = N_TOKENS // NW
PART_W = 6 * LANES

BLK_A = 2048
BLK_C = 1024


def _logits_body(q_ref, wg_ref, bg_ref, lg_ref):
    lgt = lax.dot_general(wg_ref[...], q_ref[...], (((0,), (1,)), ((), ())),
                          preferred_element_type=jnp.float32)
    lg_ref[...] = lgt + bg_ref[...]


def _logits_call(query, Wg, bg2):
    nblk = N_TOKENS // BLK_A
    return pl.pallas_call(
        _logits_body,
        grid=(nblk,),
        in_specs=[
            pl.BlockSpec((BLK_A, D), lambda i: (i, 0)),
            pl.BlockSpec((D, NE), lambda i: (0, 0)),
            pl.BlockSpec((NE, 1), lambda i: (0, 0)),
        ],
        out_specs=pl.BlockSpec((NE, BLK_A), lambda i: (0, i)),
        out_shape=jax.ShapeDtypeStruct((NE, N_TOKENS), jnp.float32),
    )(query, Wg, bg2)


def _gate_body(lg_hbm, sw_hbm, part_hbm, lv, wv, pv):
    wid = lax.axis_index("s") * NC + lax.axis_index("c")
    base = wid * CHUNK
    for e in range(NE):
        pltpu.sync_copy(lg_hbm.at[pl.ds(e * N_TOKENS + base, CHUNK)],
                        lv.at[pl.ds(e * CHUNK, CHUNK)])

    fzero = jnp.zeros((LANES,), jnp.float32)
    fone = jnp.full((LANES,), 1.0, jnp.float32)

    def step(j, acc):
        i0, i1, i2, c0, c1, c2 = acc
        off = j * LANES
        l0 = lv[pl.ds(off, LANES)]
        l1 = lv[pl.ds(CHUNK + off, LANES)]
        l2 = lv[pl.ds(2 * CHUNK + off, LANES)]

        j2 = (l2 <= l0) & (l2 <= l1)
        j1 = (~j2) & (l1 <= l0)
        j0 = ~(j2 | j1)

        m = jnp.maximum(l0, jnp.maximum(l1, l2))
        e0 = jnp.exp(l0 - m)
        e1 = jnp.exp(l1 - m)
        e2 = jnp.exp(l2 - m)
        rt = fone / (e0 + e1 + e2)

        z0 = jnp.where(j0, fzero, e0)
        z1 = jnp.where(j1, fzero, e1)
        z2 = jnp.where(j2, fzero, e2)
        rs = fone / (z0 + z1 + z2)
        wv[pl.ds(off, LANES)] = z0 * rs
        wv[pl.ds(CHUNK + off, LANES)] = z1 * rs
        wv[pl.ds(2 * CHUNK + off, LANES)] = z2 * rs

        return (i0 + e0 * rt, i1 + e1 * rt, i2 + e2 * rt,
                c0 + jnp.where(j0, fzero, fone),
                c1 + jnp.where(j1, fzero, fone),
                c2 + jnp.where(j2, fzero, fone))

    init = (fzero, fzero, fzero, fzero, fzero, fzero)
    acc = lax.fori_loop(0, CHUNK // LANES, step, init)
    for k in range(6):
        pv[pl.ds(k * LANES, LANES)] = acc[k]

    for e in range(NE):
        pltpu.sync_copy(wv.at[pl.ds(e * CHUNK, CHUNK)],
                        sw_hbm.at[pl.ds(e * N_TOKENS + base, CHUNK)])
    pltpu.sync_copy(pv, part_hbm.at[wid])


def _gate_call(lgt_flat):
    mesh = plsc.VectorSubcoreMesh(
        core_axis_name="c", subcore_axis_name="s",
        num_cores=NC, num_subcores=NS)
    f = pl.kernel(
        _gate_body,
        out_type=[
            jax.ShapeDtypeStruct((NE * N_TOKENS,), jnp.float32),
            jax.ShapeDtypeStruct((NW, PART_W), jnp.float32),
        ],
        mesh=mesh,
        scratch_types=[
            pltpu.VMEM((NE * CHUNK,), jnp.float32),
            pltpu.VMEM((NE * CHUNK,), jnp.float32),
            pltpu.VMEM((PART_W,), jnp.float32),
        ],
        compiler_params=pltpu.CompilerParams(needs_layout_passes=False),
    )
    return f(lgt_flat)


def _head_body(nblk, swt_ref, part_ref, bn_ref, ws_ref, bs_ref, wc_ref, bc_ref,
               w1_ref, b1_ref, gamma_ref, beta_ref, w2_ref, b2_ref,
               out_ref, loss_ref, e1_ref):
    i = pl.program_id(0)

    @pl.when(i == 0)
    def _prep():
        neural = bn_ref[...][None, :]
        symbolic = (0.5 * ws_ref[...] + bs_ref[...])[None, :]
        categorical = (0.5 * wc_ref[...] + bc_ref[...])[None, :]
        experts = jnp.concatenate([neural, symbolic, categorical], axis=0)
        e1_ref[...] = jnp.dot(experts, w1_ref[...],
                              preferred_element_type=jnp.float32)

        p = part_ref[...]
        inv_b = 1.0 / N_TOKENS
        loss = 0.0
        for e in range(NE):
            imp = jnp.sum(p[:, e * LANES:(e + 1) * LANES])
            cnt = jnp.sum(p[:, (NE + e) * LANES:(NE + e + 1) * LANES])
            loss += (imp * inv_b) * (cnt * inv_b)
        loss_ref[0] = NE * loss

    hpre = lax.dot_general(swt_ref[...], e1_ref[...], (((0,), (0,)), ((), ())),
                           preferred_element_type=jnp.float32)
    hpre = hpre + b1_ref[...][None, :]

    g = 0.5 * hpre * (1.0 + lax.erf(hpre * 0.7071067811865476))

    mu = jnp.mean(g, axis=1, keepdims=True)
    var = jnp.mean((g - mu) * (g - mu), axis=1, keepdims=True)
    hn = (g - mu) * lax.rsqrt(var + 1e-5)
    hn = hn * gamma_ref[...][None, :] + beta_ref[...][None, :]

    out = jnp.dot(hn.astype(jnp.bfloat16), w2_ref[...],
                  preferred_element_type=jnp.float32)
    out_ref[...] = out + b2_ref[...][None, :]


def _head_call(swt, part, bn, ws, bs, wc, bc, W1, b1, gamma, beta, W2, b2):
    nblk = N_TOKENS // BLK_C
    full = lambda shape: pl.BlockSpec(shape, lambda i: tuple(0 for _ in shape))
    return pl.pallas_call(
        functools.partial(_head_body, nblk),
        grid=(nblk,),
        in_specs=[
            pl.BlockSpec((NE, BLK_C), lambda i: (0, i)),
            full((NW, PART_W)), full((D,)), full((D,)), full((D,)),
            full((D,)), full((D,)), full((D, H)), full((H,)), full((H,)),
            full((H,)), full((H, D)), full((D,)),
        ],
        out_specs=[
            pl.BlockSpec((BLK_C, D), lambda i: (i, 0)),
            pl.BlockSpec(memory_space=pltpu.SMEM),
        ],
        out_shape=[
            jax.ShapeDtypeStruct((N_TOKENS, D), jnp.float32),
            jax.ShapeDtypeStruct((1,), jnp.float32),
        ],
        scratch_shapes=[pltpu.VMEM((NE, H), jnp.float32)],
    )(swt, part, bn, ws, bs, wc, bc, W1, b1, gamma, beta,
      W2.astype(jnp.bfloat16), b2)


def kernel(query, Wg, bg, Wn, bn, ws, bs, wc, bc, W1, b1, gamma, beta, W2, b2):
    lgt = _logits_call(query, Wg, bg.reshape(NE, 1))
    swt_flat, part = _gate_call(lgt.reshape(-1))
    swt = swt_flat.reshape(NE, N_TOKENS)
    out, loss = _head_call(swt, part, bn, ws, bs, wc, bc,
                           W1, b1, gamma, beta, W2, b2)
    return out, swt.T, loss[0]


# speedup vs baseline: 4.5769x; 1.0516x over previous
import functools

import jax
import jax.numpy as jnp
from jax import lax
from jax.experimental import pallas as pl
from jax.experimental.pallas import tpu as pltpu
from jax.experimental.pallas import tpu_sc as plsc

N_TOKENS = 32768
D = 768
H = 256
NE = 3

NC = 2
NS = 16
NW = NC * NS
LANES = 16
CHUNK = N_TOKENS // NW
PART_W = 6 * LANES

BLK_A = 2048
BLK_C = 2048


def _logits_body(q_ref, wg_ref, bg_ref, lg_ref):
    lgt = lax.dot_general(wg_ref[...], q_ref[...], (((0,), (1,)), ((), ())),
                          preferred_element_type=jnp.float32)
    lg_ref[...] = lgt + bg_ref[...]


def _logits_call(query, Wg, bg2):
    nblk = N_TOKENS // BLK_A
    return pl.pallas_call(
        _logits_body,
        grid=(nblk,),
        in_specs=[
            pl.BlockSpec((BLK_A, D), lambda i: (i, 0)),
            pl.BlockSpec((D, NE), lambda i: (0, 0)),
            pl.BlockSpec((NE, 1), lambda i: (0, 0)),
        ],
        out_specs=pl.BlockSpec((NE, BLK_A), lambda i: (0, i)),
        out_shape=jax.ShapeDtypeStruct((NE, N_TOKENS), jnp.float32),
    )(query, Wg, bg2)


def _gate_body(lg_hbm, sw_hbm, part_hbm, lv, wv, pv):
    wid = lax.axis_index("s") * NC + lax.axis_index("c")
    base = wid * CHUNK
    for e in range(NE):
        pltpu.sync_copy(lg_hbm.at[pl.ds(e * N_TOKENS + base, CHUNK)],
                        lv.at[pl.ds(e * CHUNK, CHUNK)])

    fzero = jnp.zeros((LANES,), jnp.float32)
    fone = jnp.full((LANES,), 1.0, jnp.float32)

    def step(j, acc):
        i0, i1, i2, c0, c1, c2 = acc
        off = j * LANES
        l0 = lv[pl.ds(off, LANES)]
        l1 = lv[pl.ds(CHUNK + off, LANES)]
        l2 = lv[pl.ds(2 * CHUNK + off, LANES)]

        j2 = (l2 <= l0) & (l2 <= l1)
        j1 = (~j2) & (l1 <= l0)
        j0 = ~(j2 | j1)

        m = jnp.maximum(l0, jnp.maximum(l1, l2))
        e0 = jnp.exp(l0 - m)
        e1 = jnp.exp(l1 - m)
        e2 = jnp.exp(l2 - m)
        rt = fone / (e0 + e1 + e2)

        z0 = jnp.where(j0, fzero, e0)
        z1 = jnp.where(j1, fzero, e1)
        z2 = jnp.where(j2, fzero, e2)
        rs = fone / (z0 + z1 + z2)
        wv[pl.ds(off, LANES)] = z0 * rs
        wv[pl.ds(CHUNK + off, LANES)] = z1 * rs
        wv[pl.ds(2 * CHUNK + off, LANES)] = z2 * rs

        return (i0 + e0 * rt, i1 + e1 * rt, i2 + e2 * rt,
                c0 + jnp.where(j0, fzero, fone),
                c1 + jnp.where(j1, fzero, fone),
                c2 + jnp.where(j2, fzero, fone))

    init = (fzero, fzero, fzero, fzero, fzero, fzero)
    acc = lax.fori_loop(0, CHUNK // LANES, step, init)
    for k in range(6):
        pv[pl.ds(k * LANES, LANES)] = acc[k]

    for e in range(NE):
        pltpu.sync_copy(wv.at[pl.ds(e * CHUNK, CHUNK)],
                        sw_hbm.at[pl.ds(e * N_TOKENS + base, CHUNK)])
    pltpu.sync_copy(pv, part_hbm.at[wid])


def _gate_call(lgt_flat):
    mesh = plsc.VectorSubcoreMesh(
        core_axis_name="c", subcore_axis_name="s",
        num_cores=NC, num_subcores=NS)
    f = pl.kernel(
        _gate_body,
        out_type=[
            jax.ShapeDtypeStruct((NE * N_TOKENS,), jnp.float32),
            jax.ShapeDtypeStruct((NW, PART_W), jnp.float32),
        ],
        mesh=mesh,
        scratch_types=[
            pltpu.VMEM((NE * CHUNK,), jnp.float32),
            pltpu.VMEM((NE * CHUNK,), jnp.float32),
            pltpu.VMEM((PART_W,), jnp.float32),
        ],
        compiler_params=pltpu.CompilerParams(needs_layout_passes=False),
    )
    return f(lgt_flat)


def _head_body(nblk, swt_ref, part_ref, bn_ref, ws_ref, bs_ref, wc_ref, bc_ref,
               w1_ref, b1_ref, gamma_ref, beta_ref, w2_ref, b2_ref,
               out_ref, loss_ref, e1_ref):
    i = pl.program_id(0)

    @pl.when(i == 0)
    def _prep():
        neural = bn_ref[...][None, :]
        symbolic = (0.5 * ws_ref[...] + bs_ref[...])[None, :]
        categorical = (0.5 * wc_ref[...] + bc_ref[...])[None, :]
        experts = jnp.concatenate([neural, symbolic, categorical], axis=0)
        e1_ref[...] = jnp.dot(experts, w1_ref[...],
                              preferred_element_type=jnp.float32)

        p = part_ref[...]
        inv_b = 1.0 / N_TOKENS
        loss = 0.0
        for e in range(NE):
            imp = jnp.sum(p[:, e * LANES:(e + 1) * LANES])
            cnt = jnp.sum(p[:, (NE + e) * LANES:(NE + e + 1) * LANES])
            loss += (imp * inv_b) * (cnt * inv_b)
        loss_ref[0] = NE * loss

    hpre = lax.dot_general(swt_ref[...], e1_ref[...], (((0,), (0,)), ((), ())),
                           preferred_element_type=jnp.float32)
    hpre = hpre + b1_ref[...][None, :]

    g = 0.5 * hpre * (1.0 + lax.erf(hpre * 0.7071067811865476))

    mu = jnp.mean(g, axis=1, keepdims=True)
    var = jnp.mean((g - mu) * (g - mu), axis=1, keepdims=True)
    hn = (g - mu) * lax.rsqrt(var + 1e-5)
    hn = hn * gamma_ref[...][None, :] + beta_ref[...][None, :]

    out = jnp.dot(hn.astype(jnp.bfloat16), w2_ref[...],
                  preferred_element_type=jnp.float32)
    out_ref[...] = out + b2_ref[...][None, :]


def _head_call(swt, part, bn, ws, bs, wc, bc, W1, b1, gamma, beta, W2, b2):
    nblk = N_TOKENS // BLK_C
    full = lambda shape: pl.BlockSpec(shape, lambda i: tuple(0 for _ in shape))
    return pl.pallas_call(
        functools.partial(_head_body, nblk),
        grid=(nblk,),
        in_specs=[
            pl.BlockSpec((NE, BLK_C), lambda i: (0, i)),
            full((NW, PART_W)), full((D,)), full((D,)), full((D,)),
            full((D,)), full((D,)), full((D, H)), full((H,)), full((H,)),
            full((H,)), full((H, D)), full((D,)),
        ],
        out_specs=[
            pl.BlockSpec((BLK_C, D), lambda i: (i, 0)),
            pl.BlockSpec(memory_space=pltpu.SMEM),
        ],
        out_shape=[
            jax.ShapeDtypeStruct((N_TOKENS, D), jnp.float32),
            jax.ShapeDtypeStruct((1,), jnp.float32),
        ],
        scratch_shapes=[pltpu.VMEM((NE, H), jnp.float32)],
    )(swt, part, bn, ws, bs, wc, bc, W1, b1, gamma, beta,
      W2.astype(jnp.bfloat16), b2)


def kernel(query, Wg, bg, Wn, bn, ws, bs, wc, bc, W1, b1, gamma, beta, W2, b2):
    lgt = _logits_call(query, Wg, bg.reshape(NE, 1))
    swt_flat, part = _gate_call(lgt.reshape(-1))
    swt = swt_flat.reshape(NE, N_TOKENS)
    out, loss = _head_call(swt, part, bn, ws, bs, wc, bc,
                           W1, b1, gamma, beta, W2, b2)
    return out, swt.T, loss[0]
